# baseline (speedup 1.0000x reference)
import functools

import jax
import jax.numpy as jnp
from jax import lax
from jax.experimental import pallas as pl
from jax.experimental.pallas import tpu as pltpu
from jax.experimental.pallas import tpu_sc as plsc

B, PH, PW, C = 4, 112, 112, 96
OH, OW = 224, 224

EPB = PH * PW * C
OPB = OH * OW * C
NRANGE = 4
RANGE = OPB // NRANGE
TPW = EPB // 16
CHUNK = 5376
NCHUNK = TPW // CHUNK
VPC2 = CHUNK // 32
BLK = 128
NROW = (CHUNK + 16 + BLK - 1) // BLK
ZCH = 3136
NZ = (TPW + ZCH - 1) // ZCH
TOTAL_OUT = B * OPB


def _unpool_body(
    idx_hbm, val_hbm, out_hbm, acc,
    idxb0, idxb1, valb0, valb1, cidx0, cidx1, cval0, cval1, zbuf,
    sem_i, sem_v, sem_s0, sem_s1, sem_z,
):
    idxb = [idxb0, idxb1]
    valb = [valb0, valb1]
    cidx = [cidx0, cidx1]
    cval = [cval0, cval1]
    sem_s = [sem_s0, sem_s1]
    c = lax.axis_index("c")
    s = lax.axis_index("s")
    iota = lax.iota(jnp.int32, 16)
    zvec = jnp.zeros((16,), jnp.float32)
    ones = iota < 16
    sent = iota + s * 16

    def _zb(i, _):
        zbuf[pl.ds(i * 16, 16)] = zvec
        return _

    lax.fori_loop(0, ZCH // 16, _zb, None)

    def _drain(bi, n):
        def _w(t, _):
            pltpu.make_async_copy(
                cval[bi].at[0], acc.at[cidx[bi].at[0]], sem_s[bi]
            ).wait()
            return _

        lax.fori_loop(0, n, _w, None)

    for p in range(B * 2):
        b = p // 2
        j = p % 2
        rid = 2 * c + j
        rbase = rid * RANGE
        in_base = b * EPB + s * TPW
        out_base = b * OPB + rbase + s * (RANGE // 16)

        def _load(k, bi):
            src = pl.ds(in_base + k * CHUNK, CHUNK)
            pltpu.make_async_copy(idx_hbm.at[src], idxb[bi], sem_i).start()
            pltpu.make_async_copy(val_hbm.at[src], valb[bi], sem_v).start()

        _load(0, 0)

        def _zero(k, _):
            pltpu.make_async_copy(
                zbuf, acc.at[pl.ds(s * (RANGE // 16) + k * ZCH, ZCH)],
                sem_z,
            ).start()
            return _

        def _zwait(k, _):
            pltpu.make_async_copy(
                zbuf, acc.at[pl.ds(s * (RANGE // 16), ZCH)], sem_z
            ).wait()
            return _

        lax.fori_loop(0, NZ, _zero, None)
        lax.fori_loop(0, NZ, _zwait, None)
        plsc.subcore_barrier()

        def _chunk(kk, nprev):
            nout = []
            for bi in range(2):
                k = 2 * kk + bi
                _drain(bi, nprev[bi])
                pltpu.make_async_copy(
                    idx_hbm.at[pl.ds(0, CHUNK)], idxb[bi], sem_i
                ).wait()
                pltpu.make_async_copy(
                    val_hbm.at[pl.ds(0, CHUNK)], valb[bi], sem_v
                ).wait()

                @pl.when(k + 1 < NCHUNK)
                def _():
                    _load(k + 1, 1 - bi)

                def _vec(i, cntv):
                    sl1 = pl.ds(i * 32, 16)
                    sl2 = pl.ds(i * 32 + 16, 16)
                    l1 = idxb[bi][sl1] - rbase
                    l2 = idxb[bi][sl2] - rbase
                    m1 = lax.bitcast_convert_type(l1, jnp.uint32) < jnp.uint32(RANGE)
                    m2 = lax.bitcast_convert_type(l2, jnp.uint32) < jnp.uint32(RANGE)
                    pc1 = plsc.all_reduce_population_count(m1)
                    pc2 = plsc.all_reduce_population_count(m2)
                    p1 = cntv + plsc.cumsum(m1.astype(jnp.int32)) - 1
                    p2 = cntv + pc1 + plsc.cumsum(m2.astype(jnp.int32)) - 1
                    plsc.store_scatter(cidx[bi], [p1 >> 7, p1 & 127], l1, mask=m1)
                    plsc.store_scatter(cval[bi], [p1 >> 7, p1 & 127],
                                       valb[bi][sl1], mask=m1)
                    plsc.store_scatter(cidx[bi], [p2 >> 7, p2 & 127], l2, mask=m2)
                    plsc.store_scatter(cval[bi], [p2 >> 7, p2 & 127],
                                       valb[bi][sl2], mask=m2)
                    return cntv + (pc1 + pc2)

                cntv = plsc.parallel_loop(
                    0, VPC2, unroll=7, carry=jnp.zeros((16,), jnp.int32)
                )(_vec)
                pp = cntv + iota
                plsc.store_scatter(cidx[bi], [pp >> 7, pp & 127], sent, mask=ones)
                plsc.store_scatter(cval[bi], [pp >> 7, pp & 127], zvec, mask=ones)
                cnt = jnp.max(cntv)
                cnt16 = (cnt + 15) >> 4
                nb = (cnt16 + 7) >> 3

                def _pad(g, _):
                    gp = g * 16 + iota
                    plsc.store_scatter(cidx[bi], [gp >> 7, gp & 127], sent,
                                       mask=ones)
                    plsc.store_scatter(cval[bi], [gp >> 7, gp & 127], zvec,
                                       mask=ones)
                    return _

                lax.fori_loop(cnt16, nb * 8, _pad, None)

                def _fire(t, _):
                    pltpu.async_copy(
                        cval[bi].at[t], acc.at[cidx[bi].at[t]],
                        sem_s[bi], add=True,
                    )
                    return _

                lax.fori_loop(0, nb, _fire, None)
                nout.append(nb)
            return tuple(nout)

        nlast = lax.fori_loop(
            0, NCHUNK // 2, _chunk, (jnp.int32(0), jnp.int32(0))
        )
        _drain(0, nlast[0])
        _drain(1, nlast[1])
        plsc.subcore_barrier()

        pltpu.sync_copy(
            acc.at[pl.ds(s * (RANGE // 16), RANGE // 16)],
            out_hbm.at[pl.ds(out_base, RANGE // 16)],
        )
        plsc.subcore_barrier()


@functools.cache
def _unpool():
    mesh = plsc.VectorSubcoreMesh(core_axis_name="c", subcore_axis_name="s")
    return pl.kernel(
        _unpool_body,
        out_type=jax.ShapeDtypeStruct((TOTAL_OUT,), jnp.float32),
        mesh=mesh,
        compiler_params=pltpu.CompilerParams(needs_layout_passes=False),
        scratch_types=[
            pltpu.VMEM_SHARED((RANGE,), jnp.float32),
            pltpu.VMEM((CHUNK,), jnp.int32),
            pltpu.VMEM((CHUNK,), jnp.int32),
            pltpu.VMEM((CHUNK,), jnp.float32),
            pltpu.VMEM((CHUNK,), jnp.float32),
            pltpu.VMEM((NROW, BLK), jnp.int32),
            pltpu.VMEM((NROW, BLK), jnp.int32),
            pltpu.VMEM((NROW, BLK), jnp.float32),
            pltpu.VMEM((NROW, BLK), jnp.float32),
            pltpu.VMEM((ZCH,), jnp.float32),
            pltpu.SemaphoreType.DMA,
            pltpu.SemaphoreType.DMA,
            pltpu.SemaphoreType.DMA,
            pltpu.SemaphoreType.DMA,
            pltpu.SemaphoreType.DMA,
        ],
    )


def kernel(inputs, indices, output_shape):
    idx_flat = indices.reshape(-1).astype(jnp.int32)
    val_flat = inputs.reshape(-1)
    out = _unpool()(idx_flat, val_flat)
    return out.reshape(B, OH, OW, C)


# speedup vs baseline: 1.0560x; 1.0560x over previous
import functools

import jax
import jax.numpy as jnp
from jax import lax
from jax.experimental import pallas as pl
from jax.experimental.pallas import tpu as pltpu
from jax.experimental.pallas import tpu_sc as plsc

B, PH, PW, C = 4, 112, 112, 96
OH, OW = 224, 224

EPB = PH * PW * C
OPB = OH * OW * C
NRANGE = 4
RANGE = OPB // NRANGE
TPW = EPB // 16
CHUNK = 5376
NCHUNK = TPW // CHUNK
VPC2 = CHUNK // 32
BLK = 128
NROW = (CHUNK + 16 + BLK - 1) // BLK
ZCH = 1568
NZ = (TPW + ZCH - 1) // ZCH
TOTAL_OUT = B * OPB


def _unpool_body(
    idx_hbm, val_hbm, out_hbm, acc,
    idxb0, idxb1, valb0, valb1, cidx0, cidx1, cval0, cval1, zbuf,
    sem_i, sem_v, sem_s0, sem_s1, sem_z,
):
    idxb = [idxb0, idxb1]
    valb = [valb0, valb1]
    cidx = [cidx0, cidx1]
    cval = [cval0, cval1]
    sem_s = [sem_s0, sem_s1]
    c = lax.axis_index("c")
    s = lax.axis_index("s")
    iota = lax.iota(jnp.int32, 16)
    zvec = jnp.zeros((16,), jnp.float32)
    ones = iota < 16
    sent = iota + s * 16

    def _zb(i, _):
        zbuf[pl.ds(i * 16, 16)] = zvec
        return _

    lax.fori_loop(0, ZCH // 16, _zb, None)

    def _drain(bi, n):
        def _w(t, _):
            pltpu.make_async_copy(
                cval[bi].at[0], acc.at[cidx[bi].at[0]], sem_s[bi]
            ).wait()
            return _

        lax.fori_loop(0, n, _w, None)

    for p in range(B * 2):
        b = p // 2
        j = p % 2
        rid = 2 * c + j
        rbase = rid * RANGE
        out_base = b * OPB + rbase + s * (RANGE // 16)

        def _load(k, bi):
            ph = s * 7 + (k >> 1)
            pw0 = (k & 1) * 56
            pltpu.make_async_copy(
                idx_hbm.at[b, ph, pl.ds(pw0, 56), :], idxb[bi], sem_i
            ).start()
            pltpu.make_async_copy(
                val_hbm.at[b, ph, pl.ds(pw0, 56), :], valb[bi], sem_v
            ).start()

        _load(0, 0)

        def _zero(k, _):
            pltpu.make_async_copy(
                zbuf, acc.at[pl.ds(s * (RANGE // 16) + k * ZCH, ZCH)],
                sem_z,
            ).start()
            return _

        def _zwait(k, _):
            pltpu.make_async_copy(
                zbuf, acc.at[pl.ds(s * (RANGE // 16), ZCH)], sem_z
            ).wait()
            return _

        lax.fori_loop(0, NZ, _zero, None)
        lax.fori_loop(0, NZ, _zwait, None)
        plsc.subcore_barrier()

        def _chunk(kk, nprev):
            nout = []
            for bi in range(2):
                k = 2 * kk + bi
                _drain(bi, nprev[bi])
                pltpu.make_async_copy(
                    idx_hbm.at[0, 0, pl.ds(0, 56), :], idxb[bi], sem_i
                ).wait()
                pltpu.make_async_copy(
                    val_hbm.at[0, 0, pl.ds(0, 56), :], valb[bi], sem_v
                ).wait()

                @pl.when(k + 1 < NCHUNK)
                def _():
                    _load(k + 1, 1 - bi)

                def _vec(i, cntv):
                    w = (i * 21846) >> 16
                    cc = (i - w * 3) * 32
                    sl1 = pl.ds(cc, 16)
                    sl2 = pl.ds(cc + 16, 16)
                    l1 = idxb[bi][w, sl1] - rbase
                    l2 = idxb[bi][w, sl2] - rbase
                    m1 = lax.bitcast_convert_type(l1, jnp.uint32) < jnp.uint32(RANGE)
                    m2 = lax.bitcast_convert_type(l2, jnp.uint32) < jnp.uint32(RANGE)
                    pc1 = plsc.all_reduce_population_count(m1)
                    pc2 = plsc.all_reduce_population_count(m2)
                    p1 = cntv + plsc.cumsum(m1.astype(jnp.int32)) - 1
                    p2 = cntv + pc1 + plsc.cumsum(m2.astype(jnp.int32)) - 1
                    plsc.store_scatter(cidx[bi], [p1 >> 7, p1 & 127], l1, mask=m1)
                    plsc.store_scatter(cval[bi], [p1 >> 7, p1 & 127],
                                       valb[bi][w, sl1], mask=m1)
                    plsc.store_scatter(cidx[bi], [p2 >> 7, p2 & 127], l2, mask=m2)
                    plsc.store_scatter(cval[bi], [p2 >> 7, p2 & 127],
                                       valb[bi][w, sl2], mask=m2)
                    return cntv + (pc1 + pc2)

                cntv = plsc.parallel_loop(
                    0, VPC2, unroll=7, carry=jnp.zeros((16,), jnp.int32)
                )(_vec)
                pp = cntv + iota
                plsc.store_scatter(cidx[bi], [pp >> 7, pp & 127], sent, mask=ones)
                plsc.store_scatter(cval[bi], [pp >> 7, pp & 127], zvec, mask=ones)
                cnt = jnp.max(cntv)
                cnt16 = (cnt + 15) >> 4
                nb = (cnt16 + 7) >> 3

                def _pad(g, _):
                    gp = g * 16 + iota
                    plsc.store_scatter(cidx[bi], [gp >> 7, gp & 127], sent,
                                       mask=ones)
                    plsc.store_scatter(cval[bi], [gp >> 7, gp & 127], zvec,
                                       mask=ones)
                    return _

                lax.fori_loop(cnt16, nb * 8, _pad, None)

                def _fire(t, _):
                    pltpu.async_copy(
                        cval[bi].at[t], acc.at[cidx[bi].at[t]],
                        sem_s[bi], add=True,
                    )
                    return _

                lax.fori_loop(0, nb, _fire, None)
                nout.append(nb)
            return tuple(nout)

        nlast = lax.fori_loop(
            0, NCHUNK // 2, _chunk, (jnp.int32(0), jnp.int32(0))
        )
        _drain(0, nlast[0])
        _drain(1, nlast[1])
        plsc.subcore_barrier()

        pltpu.sync_copy(
            acc.at[pl.ds(s * (RANGE // 16), RANGE // 16)],
            out_hbm.at[pl.ds(out_base, RANGE // 16)],
        )
        plsc.subcore_barrier()


@functools.cache
def _unpool():
    mesh = plsc.VectorSubcoreMesh(core_axis_name="c", subcore_axis_name="s")
    return pl.kernel(
        _unpool_body,
        out_type=jax.ShapeDtypeStruct((TOTAL_OUT,), jnp.float32),
        mesh=mesh,
        compiler_params=pltpu.CompilerParams(needs_layout_passes=False),
        scratch_types=[
            pltpu.VMEM_SHARED((RANGE,), jnp.float32),
            pltpu.VMEM((56, 96), jnp.int32),
            pltpu.VMEM((56, 96), jnp.int32),
            pltpu.VMEM((56, 96), jnp.float32),
            pltpu.VMEM((56, 96), jnp.float32),
            pltpu.VMEM((NROW, BLK), jnp.int32),
            pltpu.VMEM((NROW, BLK), jnp.int32),
            pltpu.VMEM((NROW, BLK), jnp.float32),
            pltpu.VMEM((NROW, BLK), jnp.float32),
            pltpu.VMEM((ZCH,), jnp.float32),
            pltpu.SemaphoreType.DMA,
            pltpu.SemaphoreType.DMA,
            pltpu.SemaphoreType.DMA,
            pltpu.SemaphoreType.DMA,
            pltpu.SemaphoreType.DMA,
        ],
    )


def kernel(inputs, indices, output_shape):
    out = _unpool()(indices.astype(jnp.int32), inputs)
    return out.reshape(B, OH, OW, C)
